# baseline (speedup 1.0000x reference)
import functools

import jax
import jax.numpy as jnp
from jax import lax
from jax.experimental import pallas as pl
from jax.experimental.pallas import tpu as pltpu
from jax.experimental.pallas import tpu_sc as plsc

BATCH = 1024
SEQ = 20
TOK = BATCH * SEQ
EMBED = 32
HIDDEN = 64
OUT = 1000

NUM_CORES = 2
NUM_SUBCORES = 16
NW = NUM_CORES * NUM_SUBCORES
RPW = TOK // NW
CHUNK = 128
NCHUNK = RPW // CHUNK

_sc_mesh = plsc.VectorSubcoreMesh(core_axis_name="c", subcore_axis_name="s")


@functools.partial(
    pl.kernel,
    mesh=_sc_mesh,
    out_type=[
        jax.ShapeDtypeStruct((TOK, EMBED), jnp.float32),
        jax.ShapeDtypeStruct((TOK, EMBED), jnp.float32),
    ],
    scratch_types=[
        pltpu.VMEM((NCHUNK, CHUNK), jnp.int32),
        pltpu.VMEM((NCHUNK, CHUNK), jnp.int32),
        pltpu.VMEM((RPW, EMBED), jnp.float32),
        pltpu.VMEM((RPW, EMBED), jnp.float32),
        pltpu.SemaphoreType.DMA,
        pltpu.SemaphoreType.DMA,
    ],
    compiler_params=pltpu.CompilerParams(use_tc_tiling_on_sc=False),
)
def _sc_gather(vid_hbm, pid_hbm, vtab_hbm, ptab_hbm, vout_hbm, pout_hbm,
               vidx, pidx, vrows, prows, vsem, psem):
    wid = lax.axis_index("s") * NUM_CORES + lax.axis_index("c")
    base = wid * RPW
    pltpu.sync_copy(vid_hbm.at[wid], vidx)
    pltpu.sync_copy(pid_hbm.at[wid], pidx)
    copies = []
    for j in range(NCHUNK):
        copies.append(pltpu.async_copy(
            vtab_hbm.at[vidx.at[j]], vrows.at[pl.ds(j * CHUNK, CHUNK)], vsem))
        copies.append(pltpu.async_copy(
            ptab_hbm.at[pidx.at[j]], prows.at[pl.ds(j * CHUNK, CHUNK)], psem))
    for c in copies:
        c.wait()
    pltpu.sync_copy(vrows, vout_hbm.at[pl.ds(base, RPW)])
    pltpu.sync_copy(prows, pout_hbm.at[pl.ds(base, RPW)])


TILE = 256


def _mlp_body(ve_ref, pe_ref, w1a_ref, w1b_ref, b1_ref, w3_ref, b3_ref, out_ref):
    h = jnp.dot(ve_ref[...], w1a_ref[...], preferred_element_type=jnp.float32)
    h = h + jnp.dot(pe_ref[...], w1b_ref[...], preferred_element_type=jnp.float32)
    h = jnp.maximum(h + b1_ref[...], 0.0)
    logits = jnp.dot(h, w3_ref[...], preferred_element_type=jnp.float32) + b3_ref[...]
    m = jnp.max(logits, axis=1, keepdims=True)
    e = jnp.exp(logits - m)
    lse = jnp.log(jnp.sum(e, axis=1, keepdims=True)) + m
    out_ref[...] = logits - lse


def _mlp(ve, pe, w1a, w1b, b1, w3, b3):
    return pl.pallas_call(
        _mlp_body,
        grid=(TOK // TILE,),
        in_specs=[
            pl.BlockSpec((TILE, EMBED), lambda i: (i, 0)),
            pl.BlockSpec((TILE, EMBED), lambda i: (i, 0)),
            pl.BlockSpec((EMBED, HIDDEN), lambda i: (0, 0)),
            pl.BlockSpec((EMBED, HIDDEN), lambda i: (0, 0)),
            pl.BlockSpec((1, HIDDEN), lambda i: (0, 0)),
            pl.BlockSpec((HIDDEN, OUT), lambda i: (0, 0)),
            pl.BlockSpec((1, OUT), lambda i: (0, 0)),
        ],
        out_specs=pl.BlockSpec((TILE, OUT), lambda i: (i, 0)),
        out_shape=jax.ShapeDtypeStruct((TOK, OUT), jnp.float32),
        compiler_params=pltpu.CompilerParams(dimension_semantics=("parallel",)),
    )(ve, pe, w1a, w1b, b1, w3, b3)


def kernel(vessel_ids, port_ids, vessel_table, port_table, W1, b1, W3, b3):
    vids = vessel_ids.reshape(NW, NCHUNK, CHUNK).astype(jnp.int32)
    pids = port_ids.reshape(NW, NCHUNK, CHUNK).astype(jnp.int32)
    ve, pe = _sc_gather(vids, pids, vessel_table, port_table)
    out = _mlp(ve, pe, W1[:EMBED], W1[EMBED:], b1.reshape(1, HIDDEN),
               W3, b3.reshape(1, OUT))
    return out.reshape(BATCH, SEQ, OUT)


# speedup vs baseline: 1.3315x; 1.3315x over previous
import functools

import jax
import jax.numpy as jnp
from jax import lax
from jax.experimental import pallas as pl
from jax.experimental.pallas import tpu as pltpu
from jax.experimental.pallas import tpu_sc as plsc

BATCH = 1024
SEQ = 20
TOK = BATCH * SEQ
EMBED = 32
HIDDEN = 64
OUT = 1000
VDIM = 1000000

NUM_CORES = 2
NUM_SUBCORES = 16
NW = NUM_CORES * NUM_SUBCORES
RPW = TOK // NW
CHUNK = 128
NCHUNK = RPW // CHUNK
EGROUP = 8

_sc_mesh = plsc.VectorSubcoreMesh(core_axis_name="c", subcore_axis_name="s")


@functools.partial(
    pl.kernel,
    mesh=_sc_mesh,
    out_type=jax.ShapeDtypeStruct((TOK, EMBED), jnp.float32),
    scratch_types=[
        pltpu.VMEM((NCHUNK, CHUNK), jnp.int32),
        pltpu.VMEM((RPW, EMBED), jnp.float32),
        pltpu.SemaphoreType.DMA,
    ],
    compiler_params=pltpu.CompilerParams(use_tc_tiling_on_sc=False),
)
def _sc_gather(vid_hbm, vtab_hbm, out_hbm, vidx, rows, sem):
    wid = lax.axis_index("s") * NUM_CORES + lax.axis_index("c")
    base = wid * RPW
    pltpu.sync_copy(vid_hbm.at[wid], vidx)
    copies = []
    for j in range(NCHUNK):
        copies.append(pltpu.async_copy(
            vtab_hbm.at[vidx.at[j]],
            rows.at[pl.ds(j * CHUNK, CHUNK)],
            sem))
    for c in copies:
        c.wait()
    pltpu.sync_copy(rows, out_hbm.at[pl.ds(base, RPW)])


def _mlp_body(ve_ref, pid_ref, pt_ref, w1v_ref, w1p_ref, b1_ref, w3_ref,
              b3_ref, out_ref):
    pid = pid_ref[0]
    row_ids = lax.broadcasted_iota(jnp.int32, (OUT, BATCH), 0)
    onehot = jnp.where(row_ids == pid, 1.0, 0.0).astype(jnp.float32)
    pe = lax.dot_general(pt_ref[...], onehot, (((1,), (0,)), ((), ())),
                         preferred_element_type=jnp.float32)
    h = lax.dot_general(w1v_ref[...], ve_ref[...], (((0,), (1,)), ((), ())),
                        preferred_element_type=jnp.float32)
    h = h + lax.dot_general(w1p_ref[...], pe, (((0,), (0,)), ((), ())),
                            preferred_element_type=jnp.float32)
    h = jnp.maximum(h + b1_ref[...], 0.0)
    logits = lax.dot_general(w3_ref[...], h, (((0,), (0,)), ((), ())),
                             preferred_element_type=jnp.float32)
    logits = logits + b3_ref[...]
    m = jnp.max(logits, axis=0, keepdims=True)
    e = jnp.exp(logits - m)
    lse = jnp.log(jnp.sum(e, axis=0, keepdims=True)) + m
    out_ref[0] = logits - lse


def _mlp(ve_t, pids3, pt_t, w1v, w1p, b1c, w3, b3c):
    return pl.pallas_call(
        _mlp_body,
        grid=(SEQ,),
        in_specs=[
            pl.BlockSpec((BATCH, EMBED), lambda i: (i, 0)),
            pl.BlockSpec((1, 1, BATCH), lambda i: (i, 0, 0)),
            pl.BlockSpec((EMBED, OUT), lambda i: (0, 0)),
            pl.BlockSpec((EMBED, HIDDEN), lambda i: (0, 0)),
            pl.BlockSpec((EMBED, HIDDEN), lambda i: (0, 0)),
            pl.BlockSpec((HIDDEN, 1), lambda i: (0, 0)),
            pl.BlockSpec((HIDDEN, OUT), lambda i: (0, 0)),
            pl.BlockSpec((OUT, 1), lambda i: (0, 0)),
        ],
        out_specs=pl.BlockSpec((1, OUT, BATCH), lambda i: (i, 0, 0)),
        out_shape=jax.ShapeDtypeStruct((SEQ, OUT, BATCH), jnp.float32),
        compiler_params=pltpu.CompilerParams(
            dimension_semantics=("parallel",)),
    )(ve_t, pids3, pt_t, w1v, w1p, b1c, w3, b3c)


def kernel(vessel_ids, port_ids, vessel_table, port_table, W1, b1, W3, b3):
    vids3 = vessel_ids.T.reshape(NW, NCHUNK, CHUNK).astype(jnp.int32)
    pids3 = port_ids.T.reshape(SEQ, 1, BATCH).astype(jnp.int32)
    pt_t = port_table.T
    ve = _sc_gather(vids3, vessel_table)
    out_t = _mlp(ve, pids3, pt_t, W1[:EMBED], W1[EMBED:],
                 b1.reshape(HIDDEN, 1), W3, b3.reshape(OUT, 1))
    return out_t.transpose(2, 0, 1)
